# Initial kernel scaffold; baseline (speedup 1.0000x reference)
import functools

import jax
import jax.numpy as jnp
from jax import lax
from jax.experimental import pallas as pl
from jax.experimental.pallas import tpu as pltpu
from jax.experimental.pallas import tpu_sc as plsc

VOCAB = 1000000
D = 32
B = 16384
F = 26
N = B * F

NC = 2
NS = 16
NW = NC * NS
ROWS_PER_W = N // NW

CHUNK = 128
GROUP = 8
ROWS_PER_GROUP = CHUNK * GROUP
N_GROUPS = ROWS_PER_W // ROWS_PER_GROUP

_mesh = plsc.VectorSubcoreMesh(
    core_axis_name="c", subcore_axis_name="s", num_cores=NC, num_subcores=NS
)


@functools.partial(
    pl.kernel,
    out_type=jax.ShapeDtypeStruct((N, D), jnp.float32),
    mesh=_mesh,
    scratch_types=[
        pltpu.VMEM((ROWS_PER_W,), jnp.int32),
        pltpu.VMEM((ROWS_PER_GROUP, D), jnp.float32),
        pltpu.SemaphoreType.DMA,
    ],
)
def _embed_kernel(x_hbm, w_hbm, out_hbm, idx_v, rows_v, gsem):
    wid = lax.axis_index("s") * NC + lax.axis_index("c")
    base = wid * ROWS_PER_W
    pltpu.sync_copy(x_hbm.at[pl.ds(base, ROWS_PER_W)], idx_v)

    @pl.loop(0, N_GROUPS)
    def _group(g):
        goff = g * ROWS_PER_GROUP
        copies = []
        for j in range(GROUP):
            cp = pltpu.async_copy(
                w_hbm.at[idx_v.at[pl.ds(goff + j * CHUNK, CHUNK)]],
                rows_v.at[pl.ds(j * CHUNK, CHUNK)],
                gsem,
            )
            copies.append(cp)
        for cp in copies:
            cp.wait()
        pltpu.sync_copy(rows_v, out_hbm.at[pl.ds(base + goff, ROWS_PER_GROUP)])


def kernel(x, weight):
    x_flat = x.reshape(-1).astype(jnp.int32)
    out = _embed_kernel(x_flat, weight)
    return out.reshape(B, F, D)


# speedup vs baseline: 1.5601x; 1.5601x over previous
import functools

import jax
import jax.numpy as jnp
from jax import lax
from jax.experimental import pallas as pl
from jax.experimental.pallas import tpu as pltpu
from jax.experimental.pallas import tpu_sc as plsc

VOCAB = 1000000
D = 32
B = 16384
F = 26
N = B * F

NC = 2
NS = 16
NW = NC * NS
ROWS_PER_W = N // NW

CHUNK = 128
GROUP = 8
ROWS_PER_GROUP = CHUNK * GROUP
N_GROUPS = ROWS_PER_W // ROWS_PER_GROUP

_mesh = plsc.VectorSubcoreMesh(
    core_axis_name="c", subcore_axis_name="s", num_cores=NC, num_subcores=NS
)


@functools.partial(
    pl.kernel,
    out_type=jax.ShapeDtypeStruct((N, D), jnp.float32),
    mesh=_mesh,
    scratch_types=[
        pltpu.VMEM((ROWS_PER_W,), jnp.int32),
        pltpu.VMEM((ROWS_PER_GROUP, D), jnp.float32),
        pltpu.SemaphoreType.DMA,
    ],
    compiler_params=pltpu.CompilerParams(use_tc_tiling_on_sc=False),
)
def _embed_kernel(x_hbm, w_hbm, out_hbm, idx_v, rows_v, gsem):
    wid = lax.axis_index("s") * NC + lax.axis_index("c")
    base = wid * ROWS_PER_W
    pltpu.sync_copy(x_hbm.at[pl.ds(base, ROWS_PER_W)], idx_v)

    @pl.loop(0, N_GROUPS)
    def _group(g):
        goff = g * ROWS_PER_GROUP
        copies = []
        for j in range(GROUP):
            cp = pltpu.async_copy(
                w_hbm.at[idx_v.at[pl.ds(goff + j * CHUNK, CHUNK)]],
                rows_v.at[pl.ds(j * CHUNK, CHUNK)],
                gsem,
            )
            copies.append(cp)
        for cp in copies:
            cp.wait()
        pltpu.sync_copy(rows_v, out_hbm.at[pl.ds(base + goff, ROWS_PER_GROUP)])


def kernel(x, weight):
    x_flat = x.reshape(-1).astype(jnp.int32)
    out = _embed_kernel(x_flat, weight)
    return out.reshape(B, F, D)


# speedup vs baseline: 1.5683x; 1.0053x over previous
import functools

import jax
import jax.numpy as jnp
from jax import lax
from jax.experimental import pallas as pl
from jax.experimental.pallas import tpu as pltpu
from jax.experimental.pallas import tpu_sc as plsc

VOCAB = 1000000
D = 32
B = 16384
F = 26
N = B * F

NC = 2
NS = 16
NW = NC * NS
ROWS_PER_W = N // NW

N_GROUPS = 8
ROWS_PER_GROUP = ROWS_PER_W // N_GROUPS

_mesh = plsc.VectorSubcoreMesh(
    core_axis_name="c", subcore_axis_name="s", num_cores=NC, num_subcores=NS
)


@functools.partial(
    pl.kernel,
    out_type=jax.ShapeDtypeStruct((N, D), jnp.float32),
    mesh=_mesh,
    scratch_types=[
        pltpu.VMEM((ROWS_PER_W,), jnp.int32),
        pltpu.VMEM((ROWS_PER_GROUP, D), jnp.float32),
        pltpu.VMEM((ROWS_PER_GROUP, D), jnp.float32),
        pltpu.SemaphoreType.DMA,
        pltpu.SemaphoreType.DMA,
        pltpu.SemaphoreType.DMA,
        pltpu.SemaphoreType.DMA,
    ],
    compiler_params=pltpu.CompilerParams(use_tc_tiling_on_sc=False),
)
def _embed_kernel(x_hbm, w_hbm, out_hbm, idx_v, rows0, rows1, g0, g1, s0, s1):
    wid = lax.axis_index("s") * NC + lax.axis_index("c")
    base = wid * ROWS_PER_W
    pltpu.sync_copy(x_hbm.at[pl.ds(base, ROWS_PER_W)], idx_v)

    bufs = (rows0, rows1)
    gsems = (g0, g1)
    ssems = (s0, s1)

    def fire_gather(g, slot):
        return pltpu.async_copy(
            w_hbm.at[idx_v.at[pl.ds(g * ROWS_PER_GROUP, ROWS_PER_GROUP)]],
            bufs[slot],
            gsems[slot],
        )

    def fire_store(g, slot):
        return pltpu.async_copy(
            bufs[slot],
            out_hbm.at[pl.ds(base + g * ROWS_PER_GROUP, ROWS_PER_GROUP)],
            ssems[slot],
        )

    gath = [fire_gather(0, 0), None]
    stor = [None, None]
    for g in range(N_GROUPS):
        s = g & 1
        s2 = s ^ 1
        gath[s].wait()
        if g + 1 < N_GROUPS:
            if stor[s2] is not None:
                stor[s2].wait()
            gath[s2] = fire_gather(g + 1, s2)
        stor[s] = fire_store(g, s)
    stor[0].wait()
    stor[1].wait()


def kernel(x, weight):
    x_flat = x.reshape(-1).astype(jnp.int32)
    out = _embed_kernel(x_flat, weight)
    return out.reshape(B, F, D)
